# baseline (speedup 1.0000x reference)
import jax
import jax.numpy as jnp
from jax import lax
from jax.experimental import pallas as pl
from jax.experimental.pallas import tpu as pltpu
from jax.experimental.pallas import tpu_sc as plsc

_HIDDEN = 1024
_EXPERTS = 64
_TOKENS = 32768
_BLK = 4096

_NC = 2
_NS = 16
_NW = _NC * _NS
_TPW = _TOKENS // _NW
_CH = 256
_NCHUNK = _TPW // _CH


def _matmul_block(h_ref, w_ref, logits_ref):
    logits_ref[...] = jnp.dot(
        h_ref[...], w_ref[...], preferred_element_type=jnp.float32
    )


def _sc_top2(logitsT_hbm, w1_hbm, i1_hbm, i2_hbm, chunk_v, w1_v, i1_v, i2_v):
    wid = lax.axis_index("s") * _NC + lax.axis_index("c")
    zeros = jnp.zeros((16,), jnp.int32)
    neg_inf = jnp.full((16,), -jnp.inf, jnp.float32)

    for c in range(_NCHUNK):
        t0 = wid * _TPW + c * _CH

        def copy_body(e, _):
            pltpu.sync_copy(
                logitsT_hbm.at[pl.ds(e * _TOKENS + t0, _CH)],
                chunk_v.at[pl.ds(e * _CH, _CH)],
            )
            return 0

        lax.fori_loop(0, _EXPERTS, copy_body, 0)

        def group_body(g, _):
            o = g * 16

            def e_body(e, carry):
                m1, i1, m2, i2 = carry
                ei = zeros + e
                v = chunk_v[pl.ds(e * _CH + o, 16)]
                gt1 = v > m1
                gt2 = v > m2
                nm2 = jnp.where(gt1, m1, jnp.where(gt2, v, m2))
                ni2 = jnp.where(gt1, i1, jnp.where(gt2, ei, i2))
                nm1 = jnp.where(gt1, v, m1)
                ni1 = jnp.where(gt1, ei, i1)
                return nm1, ni1, nm2, ni2

            m1, i1, m2, i2 = lax.fori_loop(
                0, _EXPERTS, e_body, (neg_inf, zeros, neg_inf, zeros)
            )
            e = jnp.exp(m2 - m1)
            w1_v[pl.ds(o, 16)] = 1.0 / (1.0 + e)
            i1_v[pl.ds(o, 16)] = i1
            i2_v[pl.ds(o, 16)] = i2
            return 0

        lax.fori_loop(0, _CH // 16, group_body, 0)
        pltpu.sync_copy(w1_v, w1_hbm.at[pl.ds(t0, _CH)])
        pltpu.sync_copy(i1_v, i1_hbm.at[pl.ds(t0, _CH)])
        pltpu.sync_copy(i2_v, i2_hbm.at[pl.ds(t0, _CH)])


def kernel(hidden_states, gate_weight):
    wt = gate_weight.T
    logits = pl.pallas_call(
        _matmul_block,
        grid=(_TOKENS // _BLK,),
        in_specs=[
            pl.BlockSpec((_BLK, _HIDDEN), lambda i: (i, 0)),
            pl.BlockSpec((_HIDDEN, _EXPERTS), lambda i: (0, 0)),
        ],
        out_specs=pl.BlockSpec((_BLK, _EXPERTS), lambda i: (i, 0)),
        out_shape=jax.ShapeDtypeStruct((_TOKENS, _EXPERTS), jnp.float32),
        compiler_params=pltpu.CompilerParams(
            dimension_semantics=("arbitrary",),
        ),
    )(hidden_states, wt)

    logits_t_flat = logits.T.reshape(_EXPERTS * _TOKENS)
    mesh = plsc.VectorSubcoreMesh(core_axis_name="c", subcore_axis_name="s")
    w1_arr, i1_arr, i2_arr = pl.kernel(
        _sc_top2,
        mesh=mesh,
        out_type=[
            jax.ShapeDtypeStruct((_TOKENS,), jnp.float32),
            jax.ShapeDtypeStruct((_TOKENS,), jnp.int32),
            jax.ShapeDtypeStruct((_TOKENS,), jnp.int32),
        ],
        scratch_types=[
            pltpu.VMEM((_CH * _EXPERTS,), jnp.float32),
            pltpu.VMEM((_CH,), jnp.float32),
            pltpu.VMEM((_CH,), jnp.int32),
            pltpu.VMEM((_CH,), jnp.int32),
        ],
    )(logits_t_flat)
    weights = jnp.stack([w1_arr, 1.0 - w1_arr], axis=1)
    idx = jnp.stack([i1_arr, i2_arr], axis=1)
    return (weights, idx, logits)


# speedup vs baseline: 2.6466x; 2.6466x over previous
import jax
import jax.numpy as jnp
from jax.experimental import pallas as pl
from jax.experimental.pallas import tpu as pltpu

_HIDDEN = 1024
_EXPERTS = 64
_TOKENS = 32768
_BLK = 4096


def _router_block(h_ref, w_ref, weights_ref, idx_ref, logits_ref):
    logits = jnp.dot(h_ref[...], w_ref[...], preferred_element_type=jnp.float32)
    logits_ref[...] = logits

    ids_f = jax.lax.broadcasted_iota(jnp.int32, logits.shape, 1).astype(jnp.float32)
    m1 = jnp.max(logits, axis=1, keepdims=True)
    f1 = jnp.where(logits == m1, 1.0, 0.0)
    i1 = jnp.sum(f1 * ids_f, axis=1, keepdims=True)
    masked = jnp.where(f1 > 0.0, -jnp.inf, logits)
    m2 = jnp.max(masked, axis=1, keepdims=True)
    f2 = jnp.where(masked == m2, 1.0, 0.0)
    i2 = jnp.sum(f2 * ids_f, axis=1, keepdims=True)

    e = jnp.exp(m2 - m1)
    w1 = 1.0 / (1.0 + e)
    weights_ref[...] = jnp.concatenate([w1, 1.0 - w1], axis=1)
    idx_ref[...] = jnp.concatenate([i1, i2], axis=1).astype(jnp.int32)


def kernel(hidden_states, gate_weight):
    wt = gate_weight.T
    grid = (_TOKENS // _BLK,)
    out = pl.pallas_call(
        _router_block,
        grid=grid,
        in_specs=[
            pl.BlockSpec((_BLK, _HIDDEN), lambda i: (i, 0)),
            pl.BlockSpec((_HIDDEN, _EXPERTS), lambda i: (0, 0)),
        ],
        out_specs=[
            pl.BlockSpec((_BLK, 2), lambda i: (i, 0)),
            pl.BlockSpec((_BLK, 2), lambda i: (i, 0)),
            pl.BlockSpec((_BLK, _EXPERTS), lambda i: (i, 0)),
        ],
        out_shape=[
            jax.ShapeDtypeStruct((_TOKENS, 2), jnp.float32),
            jax.ShapeDtypeStruct((_TOKENS, 2), jnp.int32),
            jax.ShapeDtypeStruct((_TOKENS, _EXPERTS), jnp.float32),
        ],
        compiler_params=pltpu.CompilerParams(
            dimension_semantics=("parallel",),
        ),
    )(hidden_states, wt)
    return (out[0], out[1], out[2])
